# baseline (speedup 1.0000x reference)
import functools

import jax
import jax.numpy as jnp
from jax import lax
from jax.experimental import pallas as pl
from jax.experimental.pallas import tpu as pltpu
from jax.experimental.pallas import tpu_sc as plsc

_B, _N, _D, _K = 4, 576, 64, 512
_EPS = 1e-6
_L = 16
_NC, _NS = 2, 16
_NW = _NC * _NS
_WPI = _NW // _B
_KW = _K // _WPI
_NG = _D // _L
_NR = _N // _WPI
_PSZ = 5 * _L


def _lanesum(v):
    lanes = lax.iota(jnp.int32, _L)
    for sh in (8, 4, 2, 1):
        v = v + v.at[jnp.bitwise_xor(lanes, sh)].get(mode="promise_in_bounds")
    return v


def _rsqrt_nr(x):
    xi = lax.bitcast_convert_type(x, jnp.int32)
    yi = jnp.int32(0x5F3759DF) - lax.shift_right_logical(xi, 1)
    y = lax.bitcast_convert_type(yi, jnp.float32)
    half = x * 0.5
    for _ in range(2):
        y = y * (1.5 - half * y * y)
    return y


@functools.partial(
    pl.kernel,
    out_type=(
        jax.ShapeDtypeStruct((_B, _K, _D), jnp.float32),
        jax.ShapeDtypeStruct((_B, _K), jnp.float32),
    ),
    mesh=plsc.VectorSubcoreMesh(core_axis_name="c", subcore_axis_name="s"),
    scratch_types=[
        pltpu.VMEM((_NR, _D), jnp.float32),
        pltpu.VMEM((_N,), jnp.float32),
        pltpu.VMEM((_KW, _D), jnp.float32),
        pltpu.VMEM((_KW, _D), jnp.float32),
        pltpu.VMEM((_KW,), jnp.float32),
        pltpu.VMEM((_PSZ,), jnp.float32),
        pltpu.VMEM((_WPI * _PSZ,), jnp.float32),
        pltpu.VMEM_SHARED((_NS * _PSZ,), jnp.float32),
        pltpu.SemaphoreType.DMA,
        pltpu.SemaphoreType.DMA,
    ],
)
def _grouping_sc(feat_hbm, att_hbm, cent_hbm, grouped_hbm, weights_hbm,
                 feat_v, att_v, cent_v, out_v, wout_v, part_v, comb_v,
                 shared, in_sem, cent_sem):
    c = lax.axis_index("c")
    s = lax.axis_index("s")
    w = c * _NS + s
    b = w // _WPI
    p = lax.rem(w, _WPI)

    feat_cp = pltpu.async_copy(
        feat_hbm.at[b, pl.ds(p * _NR, _NR)], feat_v, in_sem)
    att_cp = pltpu.async_copy(att_hbm.at[b], att_v, in_sem)
    cent_cp = pltpu.async_copy(
        cent_hbm.at[pl.ds(p * _KW, _KW)], cent_v, cent_sem)
    feat_cp.wait()
    att_cp.wait()

    zero = jnp.zeros((_L,), jnp.float32)
    att_base = p * _NR

    def red_step(a0, a1, a2, a3, av, j, row0):
        a = av[j]
        row = row0 + j
        a0 = a0 + a * feat_v[row, pl.ds(0, _L)]
        a1 = a1 + a * feat_v[row, pl.ds(_L, _L)]
        a2 = a2 + a * feat_v[row, pl.ds(2 * _L, _L)]
        a3 = a3 + a * feat_v[row, pl.ds(3 * _L, _L)]
        return a0, a1, a2, a3

    def red_body(g, carry):
        a0, a1, a2, a3, sv = carry
        av = att_v[pl.ds(att_base + g * _L, _L)]
        row0 = g * _L
        for j in range(_L):
            a0, a1, a2, a3 = red_step(a0, a1, a2, a3, av, j, row0)
        return (a0, a1, a2, a3, sv + av)

    v0, v1, v2, v3, s_vec = lax.fori_loop(
        0, _NR // _L, red_body, (zero, zero, zero, zero, zero))
    ntail = (_NR // _L) * _L
    av = att_v[pl.ds(att_base + ntail, _L)]
    av = jnp.where(lax.iota(jnp.int32, _L) < (_NR - ntail), av, 0.0)
    for j in range(_NR - ntail):
        v0, v1, v2, v3 = red_step(v0, v1, v2, v3, av, j, ntail)
    s_vec = s_vec + av

    part_v[pl.ds(0, _L)] = v0
    part_v[pl.ds(_L, _L)] = v1
    part_v[pl.ds(2 * _L, _L)] = v2
    part_v[pl.ds(3 * _L, _L)] = v3
    part_v[pl.ds(4 * _L, _L)] = s_vec
    pltpu.sync_copy(part_v, shared.at[pl.ds(s * _PSZ, _PSZ)])
    plsc.subcore_barrier()
    img_base = (s // _WPI) * (_WPI * _PSZ)
    pltpu.sync_copy(shared.at[pl.ds(img_base, _WPI * _PSZ)], comb_v)

    def comb(slot):
        acc = comb_v[pl.ds(slot * _L, _L)]
        for r in range(1, _WPI):
            acc = acc + comb_v[pl.ds(r * _PSZ + slot * _L, _L)]
        return acc

    v0, v1, v2, v3 = comb(0), comb(1), comb(2), comb(3)
    s_tot = _lanesum(comb(4))
    cent_cp.wait()

    def row_body(i, carry):
        d0 = v0 - s_tot * cent_v[i, pl.ds(0, _L)]
        d1 = v1 - s_tot * cent_v[i, pl.ds(_L, _L)]
        d2 = v2 - s_tot * cent_v[i, pl.ds(2 * _L, _L)]
        d3 = v3 - s_tot * cent_v[i, pl.ds(3 * _L, _L)]
        t = d0 * d0 + d1 * d1 + d2 * d2 + d3 * d3
        ssv = _lanesum(t)
        norm = ssv * _rsqrt_nr(ssv)
        scale = 1.0 / (norm + _EPS)
        out_v[i, pl.ds(0, _L)] = d0 * scale
        out_v[i, pl.ds(_L, _L)] = d1 * scale
        out_v[i, pl.ds(2 * _L, _L)] = d2 * scale
        out_v[i, pl.ds(3 * _L, _L)] = d3 * scale
        return carry

    lax.fori_loop(0, _KW, row_body, jnp.int32(0))

    wv = s_tot * (1.0 / _N)
    for j in range(_KW // _L):
        wout_v[pl.ds(j * _L, _L)] = wv

    pltpu.sync_copy(out_v, grouped_hbm.at[b, pl.ds(p * _KW, _KW)])
    pltpu.sync_copy(wout_v, weights_hbm.at[b, pl.ds(p * _KW, _KW)])


def kernel(features, attentions, centroids):
    B, N, _ = features.shape
    return _grouping_sc(features, attentions.reshape(B, N), centroids)


# speedup vs baseline: 1.0633x; 1.0633x over previous
import functools

import jax
import jax.numpy as jnp
from jax import lax
from jax.experimental import pallas as pl
from jax.experimental.pallas import tpu as pltpu
from jax.experimental.pallas import tpu_sc as plsc

_B, _N, _D, _K = 4, 576, 64, 512
_EPS = 1e-6
_L = 16
_NC, _NS = 2, 16
_NW = _NC * _NS
_WPI = _NW // _B
_KW = _K // _WPI
_NR = _N // _WPI
_PSZ = 5 * _L


def _lanesum(v):
    lanes = lax.iota(jnp.int32, _L)
    for sh in (8, 4, 2, 1):
        v = v + v.at[jnp.bitwise_xor(lanes, sh)].get(mode="promise_in_bounds")
    return v


def _rsqrt_nr(x):
    xi = lax.bitcast_convert_type(x, jnp.int32)
    yi = jnp.int32(0x5F3759DF) - lax.shift_right_logical(xi, 1)
    y = lax.bitcast_convert_type(yi, jnp.float32)
    half = x * 0.5
    for _ in range(2):
        y = y * (1.5 - half * y * y)
    return y


@functools.partial(
    pl.kernel,
    out_type=(
        jax.ShapeDtypeStruct((_B * _K * _D,), jnp.float32),
        jax.ShapeDtypeStruct((_B * _K,), jnp.float32),
    ),
    mesh=plsc.VectorSubcoreMesh(core_axis_name="c", subcore_axis_name="s"),
    scratch_types=[
        pltpu.VMEM((_NR * _D,), jnp.float32),
        pltpu.VMEM((_N + _L,), jnp.float32),
        pltpu.VMEM((_KW * _D,), jnp.float32),
        pltpu.VMEM((_KW * _D,), jnp.float32),
        pltpu.VMEM((_KW,), jnp.float32),
        pltpu.VMEM((_PSZ,), jnp.float32),
        pltpu.VMEM((_WPI * _PSZ,), jnp.float32),
        pltpu.VMEM_SHARED((_NS * _PSZ,), jnp.float32),
        pltpu.SemaphoreType.DMA,
        pltpu.SemaphoreType.DMA,
    ],
)
def _grouping_sc(feat_hbm, att_hbm, cent_hbm, grouped_hbm, weights_hbm,
                 feat_v, att_v, cent_v, out_v, wout_v, part_v, comb_v,
                 shared, in_sem, cent_sem):
    c = lax.axis_index("c")
    s = lax.axis_index("s")
    w = c * _NS + s
    b = w // _WPI
    p = lax.rem(w, _WPI)

    feat_cp = pltpu.async_copy(
        feat_hbm.at[pl.ds(b * (_N * _D) + p * (_NR * _D), _NR * _D)], feat_v,
        in_sem)
    att_cp = pltpu.async_copy(
        att_hbm.at[pl.ds(b * _N, _N)], att_v.at[pl.ds(0, _N)], in_sem)
    cent_cp = pltpu.async_copy(
        cent_hbm.at[pl.ds(p * (_KW * _D), _KW * _D)], cent_v, cent_sem)
    feat_cp.wait()
    att_cp.wait()

    zero = jnp.zeros((_L,), jnp.float32)
    zidx = jnp.zeros((_L,), jnp.int32)
    att_base = p * _NR

    def red_body(n, carry):
        a0, a1, a2, a3 = carry
        attn = att_v[pl.ds(att_base + n, _L)]
        aB = attn.at[zidx].get(mode="promise_in_bounds")
        base = n * _D
        a0 = a0 + aB * feat_v[pl.ds(base, _L)]
        a1 = a1 + aB * feat_v[pl.ds(base + _L, _L)]
        a2 = a2 + aB * feat_v[pl.ds(base + 2 * _L, _L)]
        a3 = a3 + aB * feat_v[pl.ds(base + 3 * _L, _L)]
        return (a0, a1, a2, a3)

    v0, v1, v2, v3 = lax.fori_loop(0, _NR, red_body, (zero, zero, zero, zero))

    ntail = (_NR // _L) * _L
    s_vec = zero
    for g in range(_NR // _L):
        s_vec = s_vec + att_v[pl.ds(att_base + g * _L, _L)]
    tail = att_v[pl.ds(att_base + ntail, _L)]
    s_vec = s_vec + jnp.where(lax.iota(jnp.int32, _L) < (_NR - ntail), tail,
                              0.0)

    part_v[pl.ds(0, _L)] = v0
    part_v[pl.ds(_L, _L)] = v1
    part_v[pl.ds(2 * _L, _L)] = v2
    part_v[pl.ds(3 * _L, _L)] = v3
    part_v[pl.ds(4 * _L, _L)] = s_vec
    pltpu.sync_copy(part_v, shared.at[pl.ds(s * _PSZ, _PSZ)])
    plsc.subcore_barrier()
    img_base = (s // _WPI) * (_WPI * _PSZ)
    pltpu.sync_copy(shared.at[pl.ds(img_base, _WPI * _PSZ)], comb_v)

    def comb(slot):
        acc = comb_v[pl.ds(slot * _L, _L)]
        for r in range(1, _WPI):
            acc = acc + comb_v[pl.ds(r * _PSZ + slot * _L, _L)]
        return acc

    v0, v1, v2, v3 = comb(0), comb(1), comb(2), comb(3)
    s_tot = _lanesum(comb(4))
    cent_cp.wait()

    def row_body(i, carry):
        base = i * _D
        d0 = v0 - s_tot * cent_v[pl.ds(base, _L)]
        d1 = v1 - s_tot * cent_v[pl.ds(base + _L, _L)]
        d2 = v2 - s_tot * cent_v[pl.ds(base + 2 * _L, _L)]
        d3 = v3 - s_tot * cent_v[pl.ds(base + 3 * _L, _L)]
        t = d0 * d0 + d1 * d1 + d2 * d2 + d3 * d3
        ssv = _lanesum(t)
        norm = ssv * _rsqrt_nr(ssv)
        scale = 1.0 / (norm + _EPS)
        out_v[pl.ds(base, _L)] = d0 * scale
        out_v[pl.ds(base + _L, _L)] = d1 * scale
        out_v[pl.ds(base + 2 * _L, _L)] = d2 * scale
        out_v[pl.ds(base + 3 * _L, _L)] = d3 * scale
        return carry

    lax.fori_loop(0, _KW, row_body, jnp.int32(0))

    wv = s_tot * (1.0 / _N)
    for j in range(_KW // _L):
        wout_v[pl.ds(j * _L, _L)] = wv

    out_base = (b * _K + p * _KW) * _D
    pltpu.sync_copy(out_v, grouped_hbm.at[pl.ds(out_base, _KW * _D)])
    pltpu.sync_copy(wout_v, weights_hbm.at[pl.ds(b * _K + p * _KW, _KW)])


def kernel(features, attentions, centroids):
    B, N, D = features.shape
    K = centroids.shape[0]
    g, wts = _grouping_sc(
        features.reshape(B * N * D),
        attentions.reshape(B * N),
        centroids.reshape(K * D),
    )
    return g.reshape(B, K, D), wts.reshape(B, K)
